# baseline (speedup 1.0000x reference)
import jax
import jax.numpy as jnp
from jax import lax
from jax.experimental import pallas as pl
from jax.experimental.pallas import tpu as pltpu
from jax.experimental.pallas import tpu_sc as plsc

BATCH = 64
BLOCK = 72
MAX_PATH = 5
NUM_NODES = BATCH * BLOCK
WIN = 256
CELLS = BLOCK * BLOCK
WORDS = CELLS * MAX_PATH
N_WORKERS = 32
BLOCKS_PER_WORKER = BATCH // N_WORKERS


def _win_start(i):
    return min((i * BLOCK // 128) * 128, NUM_NODES - WIN)


def _sc_body(paths_hbm, b_hbm, w_hbm, p_vmem, w_vmem, b_vmem):
    pltpu.sync_copy(b_hbm, b_vmem)
    wid = lax.axis_index("s") * 2 + lax.axis_index("c")
    for t in range(BLOCKS_PER_WORKER):
        i = wid * BLOCKS_PER_WORKER + t
        pltpu.sync_copy(paths_hbm.at[pl.ds(i * WORDS, WORDS)], p_vmem)
        start = i * BLOCK
        astart = jnp.minimum((start // 128) * 128, NUM_NODES - WIN)
        off = start - astart

        def memset_row(j, _):
            r = j // (WIN // 16)
            cc = (j % (WIN // 16)) * 16
            w_vmem[r, pl.ds(cc, 16)] = jnp.zeros((16,), jnp.float32)
            return 0

        lax.fori_loop(0, BLOCK * (WIN // 16), memset_row, 0)

        def chunk(q, _):
            e = q * 16 + lax.iota(jnp.int32, 16)
            base = e * MAX_PATH
            acc = jnp.zeros((16,), jnp.int32)
            for k in range(MAX_PATH):
                g = plsc.load_gather(p_vmem, [base + k])
                acc = acc + (g != -1).astype(jnp.int32)
            vals = plsc.load_gather(b_vmem, [acc])
            r = e // BLOCK
            c = e - r * BLOCK + off
            plsc.store_scatter(w_vmem, [r, c], vals)
            return 0

        lax.fori_loop(0, CELLS // 16, chunk, 0)
        pltpu.sync_copy(w_vmem, w_hbm.at[i])


def _sc_windows(paths_flat, b16):
    mesh = plsc.VectorSubcoreMesh(core_axis_name="c", subcore_axis_name="s")
    return pl.kernel(
        _sc_body,
        out_type=jax.ShapeDtypeStruct((BATCH, BLOCK, WIN), jnp.float32),
        mesh=mesh,
        scratch_types=[
            pltpu.VMEM((WORDS,), jnp.int32),
            pltpu.VMEM((BLOCK, WIN), jnp.float32),
            pltpu.VMEM((16,), jnp.float32),
        ],
        compiler_params=pltpu.CompilerParams(needs_layout_passes=False),
    )(paths_flat, b16)


def _tc_body(w_ref, out_ref, zeros_ref, sem):
    zeros_ref[...] = jnp.zeros((BLOCK, NUM_NODES), dtype=jnp.float32)
    copies = []
    for i in range(BATCH):
        r0, r1 = i * BLOCK, (i + 1) * BLOCK
        a = _win_start(i)
        if a > 0:
            copies.append(pltpu.make_async_copy(
                zeros_ref.at[:, :a], out_ref.at[r0:r1, :a], sem))
        if a + WIN < NUM_NODES:
            copies.append(pltpu.make_async_copy(
                zeros_ref.at[:, a + WIN:], out_ref.at[r0:r1, a + WIN:], sem))
        copies.append(pltpu.make_async_copy(
            w_ref.at[i], out_ref.at[r0:r1, a:a + WIN], sem))
    for c in copies:
        c.start()
    for c in copies:
        c.wait()


def kernel(x, paths, b):
    del x
    paths_flat = paths.astype(jnp.int32).reshape(-1)
    b16 = jnp.pad(b, (0, 16 - b.shape[0]))
    w = _sc_windows(paths_flat, b16)
    return pl.pallas_call(
        _tc_body,
        in_specs=[pl.BlockSpec(memory_space=pltpu.VMEM)],
        out_specs=pl.BlockSpec(memory_space=pl.ANY),
        out_shape=jax.ShapeDtypeStruct((NUM_NODES, NUM_NODES), jnp.float32),
        scratch_shapes=[
            pltpu.VMEM((BLOCK, NUM_NODES), jnp.float32),
            pltpu.SemaphoreType.DMA,
        ],
    )(w)


# speedup vs baseline: 1.0087x; 1.0087x over previous
import jax
import jax.numpy as jnp
from jax import lax
from jax.experimental import pallas as pl
from jax.experimental.pallas import tpu as pltpu
from jax.experimental.pallas import tpu_sc as plsc

BATCH = 64
BLOCK = 72
MAX_PATH = 5
NUM_NODES = BATCH * BLOCK
WIN = 256
CELLS = BLOCK * BLOCK
WORDS = CELLS * MAX_PATH
N_TILES = (BLOCK // 8) * (WIN // 128)
N_WORKERS = 32
BLOCKS_PER_WORKER = BATCH // N_WORKERS


def _win_start(i):
    return min((i * BLOCK // 128) * 128, NUM_NODES - WIN)


def _sc_body(paths_hbm, b_hbm, w_hbm, p_vmem, w_vmem, b_vmem):
    pltpu.sync_copy(b_hbm, b_vmem)
    wid = lax.axis_index("s") * 2 + lax.axis_index("c")
    for t in range(BLOCKS_PER_WORKER):
        i = wid * BLOCKS_PER_WORKER + t
        pltpu.sync_copy(paths_hbm.at[pl.ds(i * WORDS, WORDS)], p_vmem)
        start = i * BLOCK
        astart = jnp.minimum((start // 128) * 128, NUM_NODES - WIN)
        off = start - astart

        def memset_row(j, _):
            tt = j // (8 * (128 // 16))
            rem = j % (8 * (128 // 16))
            rs = rem // (128 // 16)
            cc = (rem % (128 // 16)) * 16
            w_vmem[tt, rs, pl.ds(cc, 16)] = jnp.zeros((16,), jnp.float32)
            return 0

        lax.fori_loop(0, N_TILES * 8 * (128 // 16), memset_row, 0)

        def chunk(q, _):
            e = q * 16 + lax.iota(jnp.int32, 16)
            r = e // BLOCK
            c = e - r * BLOCK
            acc = jnp.zeros((16,), jnp.int32)
            for k in range(MAX_PATH):
                g = plsc.load_gather(p_vmem, [e * MAX_PATH + k])
                acc = acc + (g != -1).astype(jnp.int32)
            vals = plsc.load_gather(b_vmem, [acc])
            cw = c + off
            tt = (r // 8) * 2 + cw // 128
            plsc.store_scatter(w_vmem, [tt, r % 8, cw % 128], vals)
            return 0

        lax.fori_loop(0, CELLS // 16, chunk, 0)
        pltpu.sync_copy(w_vmem, w_hbm.at[i])


def _sc_windows(paths, b16):
    mesh = plsc.VectorSubcoreMesh(core_axis_name="c", subcore_axis_name="s")
    return pl.kernel(
        _sc_body,
        out_type=jax.ShapeDtypeStruct((BATCH, N_TILES, 8, 128), jnp.float32),
        mesh=mesh,
        scratch_types=[
            pltpu.VMEM((WORDS,), jnp.int32),
            pltpu.VMEM((N_TILES, 8, 128), jnp.float32),
            pltpu.VMEM((16,), jnp.float32),
        ],
        compiler_params=pltpu.CompilerParams(needs_layout_passes=False),
    )(paths, b16)


def _tc_fill_body(out_ref, zeros_ref, sem):
    zeros_ref[...] = jnp.zeros((BLOCK, NUM_NODES), dtype=jnp.float32)
    copies = []
    for i in range(BATCH):
        r0, r1 = i * BLOCK, (i + 1) * BLOCK
        a = _win_start(i)
        if a > 0:
            copies.append(pltpu.make_async_copy(
                zeros_ref.at[:, :a], out_ref.at[r0:r1, :a], sem))
        if a + WIN < NUM_NODES:
            copies.append(pltpu.make_async_copy(
                zeros_ref.at[:, a + WIN:], out_ref.at[r0:r1, a + WIN:], sem))
    for c in copies:
        c.start()
    for c in copies:
        c.wait()


def _tc_place_body(w_ref, filled_ref, out_ref, win_scratch, sem):
    del filled_ref
    NBUF = 4
    inflight = [None] * NBUF
    for i in range(BATCH):
        buf = i % NBUF
        if inflight[buf] is not None:
            inflight[buf].wait()
        for tt in range(N_TILES):
            tr, tc = tt // 2, tt % 2
            win_scratch[buf, tr * 8:(tr + 1) * 8,
                        tc * 128:(tc + 1) * 128] = w_ref[i, tt]
        a = _win_start(i)
        cp = pltpu.make_async_copy(
            win_scratch.at[buf],
            out_ref.at[i * BLOCK:(i + 1) * BLOCK, a:a + WIN], sem)
        cp.start()
        inflight[buf] = cp
    for cp in inflight:
        cp.wait()


def kernel(x, paths, b):
    del x
    p32 = paths.astype(jnp.int32)
    b16 = jnp.pad(b, (0, 16 - b.shape[0]))
    w = _sc_windows(p32.reshape(-1), b16)
    filled = pl.pallas_call(
        _tc_fill_body,
        out_specs=pl.BlockSpec(memory_space=pl.ANY),
        out_shape=jax.ShapeDtypeStruct((NUM_NODES, NUM_NODES), jnp.float32),
        scratch_shapes=[
            pltpu.VMEM((BLOCK, NUM_NODES), jnp.float32),
            pltpu.SemaphoreType.DMA,
        ],
    )()
    return pl.pallas_call(
        _tc_place_body,
        in_specs=[
            pl.BlockSpec(memory_space=pltpu.VMEM),
            pl.BlockSpec(memory_space=pl.ANY),
        ],
        out_specs=pl.BlockSpec(memory_space=pl.ANY),
        out_shape=jax.ShapeDtypeStruct((NUM_NODES, NUM_NODES), jnp.float32),
        input_output_aliases={1: 0},
        scratch_shapes=[
            pltpu.VMEM((4, BLOCK, WIN), jnp.float32),
            pltpu.SemaphoreType.DMA,
        ],
    )(w, filled)


# speedup vs baseline: 2.0578x; 2.0401x over previous
import jax
import jax.numpy as jnp
from jax import lax
from jax.experimental import pallas as pl
from jax.experimental.pallas import tpu as pltpu
from jax.experimental.pallas import tpu_sc as plsc

BATCH = 64
BLOCK = 72
MAX_PATH = 5
NUM_NODES = BATCH * BLOCK
WIN = 256
CELLS = BLOCK * BLOCK
WORDS = CELLS * MAX_PATH
N_TILES = (BLOCK // 8) * (WIN // 128)
N_WORKERS = 32
BLOCKS_PER_WORKER = BATCH // N_WORKERS


def _win_start(i):
    return min((i * BLOCK // 128) * 128, NUM_NODES - WIN)


def _sc_body(paths_hbm, b_hbm, w_hbm, p_vmem, w_vmem, b_vmem):
    pltpu.sync_copy(b_hbm, b_vmem)
    wid = lax.axis_index("s") * 2 + lax.axis_index("c")
    for t in range(BLOCKS_PER_WORKER):
        i = wid * BLOCKS_PER_WORKER + t
        pltpu.sync_copy(paths_hbm.at[i], p_vmem)
        start = i * BLOCK
        astart = jnp.minimum((start // 128) * 128, NUM_NODES - WIN)
        off = start - astart

        def memset_row(j, _):
            tt = j // (8 * (128 // 16))
            rem = j % (8 * (128 // 16))
            rs = rem // (128 // 16)
            cc = (rem % (128 // 16)) * 16
            w_vmem[tt, rs, pl.ds(cc, 16)] = jnp.zeros((16,), jnp.float32)
            return 0

        lax.fori_loop(0, N_TILES * 8 * (128 // 16), memset_row, 0)

        def chunk(q, _):
            e = q * 16 + lax.iota(jnp.int32, 16)
            r = e // BLOCK
            c = e - r * BLOCK
            acc = jnp.zeros((16,), jnp.int32)
            for k in range(MAX_PATH):
                f = e * MAX_PATH + k
                g = plsc.load_gather(
                    p_vmem, [f // 1024, (f // 128) % 8, f % 128])
                acc = acc + (g != -1).astype(jnp.int32)
            vals = plsc.load_gather(b_vmem, [acc])
            cw = c + off
            tt = (r // 8) * 2 + cw // 128
            plsc.store_scatter(w_vmem, [tt, r % 8, cw % 128], vals)
            return 0

        lax.fori_loop(0, CELLS // 16, chunk, 0)
        pltpu.sync_copy(w_vmem, w_hbm.at[i])


def _sc_windows(paths, b16):
    mesh = plsc.VectorSubcoreMesh(core_axis_name="c", subcore_axis_name="s")
    return pl.kernel(
        _sc_body,
        out_type=jax.ShapeDtypeStruct((BATCH, N_TILES, 8, 128), jnp.float32),
        mesh=mesh,
        scratch_types=[
            pltpu.VMEM((26, 8, 128), jnp.int32),
            pltpu.VMEM((N_TILES, 8, 128), jnp.float32),
            pltpu.VMEM((16,), jnp.float32),
        ],
        compiler_params=pltpu.CompilerParams(needs_layout_passes=False),
    )(paths, b16)


def _tc_fill_body(out_ref, zeros_ref, sem):
    zeros_ref[...] = jnp.zeros((BLOCK, NUM_NODES), dtype=jnp.float32)
    copies = []
    for i in range(BATCH):
        r0, r1 = i * BLOCK, (i + 1) * BLOCK
        a = _win_start(i)
        if a > 0:
            copies.append(pltpu.make_async_copy(
                zeros_ref.at[:, :a], out_ref.at[r0:r1, :a], sem))
        if a + WIN < NUM_NODES:
            copies.append(pltpu.make_async_copy(
                zeros_ref.at[:, a + WIN:], out_ref.at[r0:r1, a + WIN:], sem))
    for c in copies:
        c.start()
    for c in copies:
        c.wait()


def _tc_place_body(w_ref, filled_ref, out_ref, win_scratch, sem):
    del filled_ref
    NBUF = 4
    inflight = [None] * NBUF
    for i in range(BATCH):
        buf = i % NBUF
        if inflight[buf] is not None:
            inflight[buf].wait()
        for tt in range(N_TILES):
            tr, tc = tt // 2, tt % 2
            win_scratch[buf, tr * 8:(tr + 1) * 8,
                        tc * 128:(tc + 1) * 128] = w_ref[i, tt]
        a = _win_start(i)
        cp = pltpu.make_async_copy(
            win_scratch.at[buf],
            out_ref.at[i * BLOCK:(i + 1) * BLOCK, a:a + WIN], sem)
        cp.start()
        inflight[buf] = cp
    for cp in inflight:
        cp.wait()


def kernel(x, paths, b):
    del x
    p32 = paths.astype(jnp.int32)
    b16 = jnp.pad(b, (0, 16 - b.shape[0]))
    pf = jnp.pad(p32.reshape(BATCH, WORDS), ((0, 0), (0, 26 * 1024 - WORDS)))
    w = _sc_windows(pf.reshape(BATCH, 26, 8, 128), b16)
    filled = pl.pallas_call(
        _tc_fill_body,
        out_specs=pl.BlockSpec(memory_space=pl.ANY),
        out_shape=jax.ShapeDtypeStruct((NUM_NODES, NUM_NODES), jnp.float32),
        scratch_shapes=[
            pltpu.VMEM((BLOCK, NUM_NODES), jnp.float32),
            pltpu.SemaphoreType.DMA,
        ],
    )()
    return pl.pallas_call(
        _tc_place_body,
        in_specs=[
            pl.BlockSpec(memory_space=pltpu.VMEM),
            pl.BlockSpec(memory_space=pl.ANY),
        ],
        out_specs=pl.BlockSpec(memory_space=pl.ANY),
        out_shape=jax.ShapeDtypeStruct((NUM_NODES, NUM_NODES), jnp.float32),
        input_output_aliases={1: 0},
        scratch_shapes=[
            pltpu.VMEM((4, BLOCK, WIN), jnp.float32),
            pltpu.SemaphoreType.DMA,
        ],
    )(w, filled)


# speedup vs baseline: 8.0546x; 3.9142x over previous
import jax
import jax.numpy as jnp
from jax.experimental import pallas as pl
from jax.experimental.pallas import tpu as pltpu

BATCH = 64
BLOCK = 72
MAX_PATH = 5
NUM_NODES = BATCH * BLOCK
BLOCKS_PER = 8
ROWS_PER = BLOCK * BLOCKS_PER
GRID = BATCH // BLOCKS_PER


def _spatial_kernel(b_ref, paths_ref, out_ref):
    g = pl.program_id(0)
    out_ref[...] = jnp.zeros((ROWS_PER, NUM_NODES), dtype=jnp.float32)
    for r in range(BLOCKS_PER):
        i = g * BLOCKS_PER + r
        p = paths_ref[r]
        lengths = jnp.sum((p != -1).astype(jnp.int32), axis=0)
        vals = jnp.zeros((BLOCK, BLOCK), dtype=jnp.float32)
        for k in range(MAX_PATH + 1):
            vals = jnp.where(lengths == k, b_ref[k], vals)
        start = i * BLOCK
        atile = jnp.minimum(start // 128, (NUM_NODES - 256) // 128)
        astart = atile * 128
        off = start - astart
        tiled4 = jnp.concatenate([vals] * 4, axis=1)
        rolled = pltpu.roll(tiled4, off % BLOCK, axis=1)
        window = rolled[:, :256]
        c = jax.lax.broadcasted_iota(jnp.int32, (BLOCK, 256), 1)
        mask = (c >= off) & (c < off + BLOCK)
        out_ref[r * BLOCK:(r + 1) * BLOCK, pl.ds(astart, 256)] = (
            jnp.where(mask, window, 0.0))


def kernel(x, paths, b):
    del x
    p32 = jnp.transpose(paths.astype(jnp.int32), (0, 3, 1, 2))
    return pl.pallas_call(
        _spatial_kernel,
        grid=(GRID,),
        in_specs=[
            pl.BlockSpec(memory_space=pltpu.SMEM),
            pl.BlockSpec((BLOCKS_PER, MAX_PATH, BLOCK, BLOCK),
                         lambda i: (i, 0, 0, 0)),
        ],
        out_specs=pl.BlockSpec((ROWS_PER, NUM_NODES), lambda i: (i, 0)),
        out_shape=jax.ShapeDtypeStruct((NUM_NODES, NUM_NODES), jnp.float32),
        compiler_params=pltpu.CompilerParams(
            dimension_semantics=("parallel",),
        ),
    )(b, p32)
